# Initial kernel scaffold; baseline (speedup 1.0000x reference)
import jax
import jax.numpy as jnp
from jax.experimental import pallas as pl


def kernel(x, edge_index, edge_type, weight, h_bias):
    raise NotImplementedError("write your pallas kernel here")


# speedup vs baseline: 12.1919x; 12.1919x over previous
import jax
import jax.numpy as jnp
from jax import lax
from jax.experimental import pallas as pl
from jax.experimental.pallas import tpu as pltpu
from jax.experimental.pallas import tpu_sc as plsc

_N = 10000
_E = 320000
_F = 128
_H = _F // 2
_R = 8

_NC = 2
_NS = 16
_EPT = _E // _NS
_CH = 80
_NCH = _EPT // _CH
_NP = 10240
_RPT = _NP // _NS
_LANES = 16


def _mm_body(x_ref, w_ref, o_ref):
    o_ref[0, 0] = jnp.dot(x_ref[...], w_ref[0, 0],
                          preferred_element_type=jnp.float32)


def _rel_transform(x, w_halves):
    bn = 2000
    return pl.pallas_call(
        _mm_body,
        grid=(_R, _N // bn, _NC),
        in_specs=[
            pl.BlockSpec((bn, _F), lambda r, i, h: (i, 0)),
            pl.BlockSpec((1, 1, _F, _H), lambda r, i, h: (h, r, 0, 0)),
        ],
        out_specs=pl.BlockSpec((1, 1, bn, _H), lambda r, i, h: (h, r, i, 0)),
        out_shape=jax.ShapeDtypeStruct((_NC, _R, _N, _H), jnp.float32),
    )(x, w_halves)


def _sc_body(xr_hbm, et_hbm, src_hbm, dst_hbm, zeros_hbm, out_hbm,
             gi_v, src_v, dst_v, rows_v, agg_s, sem):
    c = lax.axis_index("c")
    s = lax.axis_index("s")

    pltpu.sync_copy(et_hbm.at[s], gi_v)
    pltpu.sync_copy(src_hbm.at[s], src_v)
    pltpu.sync_copy(dst_hbm.at[s], dst_v)

    rows = pl.ds(s * _RPT, _RPT)
    pltpu.sync_copy(zeros_hbm.at[rows], agg_s.at[rows])

    base = c * (_R * _N)

    def _row(i, carry):
        for k in range(_CH // _LANES):
            sl = pl.ds(k * _LANES, _LANES)
            gi_v[i, sl] = gi_v[i, sl] * _N + src_v[i, sl] + base
        return carry
    lax.fori_loop(0, _NCH, _row, 0)

    plsc.subcore_barrier()

    def _chunk(j, carry):
        pltpu.async_copy(xr_hbm.at[gi_v.at[j]], rows_v, sem).wait()
        pltpu.sync_copy(rows_v, agg_s.at[dst_v.at[j]], add=True)
        return carry
    lax.fori_loop(0, _NCH, _chunk, 0)

    plsc.subcore_barrier()

    pltpu.sync_copy(agg_s.at[rows], out_hbm.at[c, rows])


_sc_scatter = pl.kernel(
    _sc_body,
    out_type=jax.ShapeDtypeStruct((_NC, _NP, _H), jnp.float32),
    mesh=plsc.VectorSubcoreMesh(core_axis_name="c", subcore_axis_name="s",
                                num_cores=_NC, num_subcores=_NS),
    scratch_types=[
        pltpu.VMEM((_NCH, _CH), jnp.int32),
        pltpu.VMEM((_NCH, _CH), jnp.int32),
        pltpu.VMEM((_NCH, _CH), jnp.int32),
        pltpu.VMEM((_CH, _H), jnp.float32),
        pltpu.VMEM_SHARED((_NP, _H), jnp.float32),
        pltpu.SemaphoreType.DMA,
    ],
    compiler_params=pltpu.CompilerParams(use_tc_tiling_on_sc=False),
)


def _ep_body(p_ref, b_ref, o_ref):
    full = jnp.concatenate([p_ref[0], p_ref[1]], axis=1)
    bias = jnp.concatenate([b_ref[0], b_ref[1]], axis=1)
    o_ref[...] = full + bias


def _epilogue(parts, bias2d):
    bn = 2000
    return pl.pallas_call(
        _ep_body,
        grid=(_N // bn,),
        in_specs=[
            pl.BlockSpec((_NC, bn, _H), lambda i: (0, i, 0)),
            pl.BlockSpec((_NC, 1, _H), lambda i: (0, 0, 0)),
        ],
        out_specs=pl.BlockSpec((bn, _F), lambda i: (i, 0)),
        out_shape=jax.ShapeDtypeStruct((_N, _F), jnp.float32),
    )(parts, bias2d)


def kernel(x, edge_index, edge_type, weight, h_bias):
    w_halves = weight.reshape(_R, _F, _NC, _H).transpose(2, 0, 1, 3)
    xr = _rel_transform(x, w_halves).reshape(_NC * _R * _N, _H)
    et = edge_type.reshape(_NS, _NCH, _CH)
    src = edge_index[0].reshape(_NS, _NCH, _CH)
    dst = edge_index[1].reshape(_NS, _NCH, _CH)
    zeros = jnp.zeros((_NP, _H), jnp.float32)
    parts = _sc_scatter(xr, et, src, dst, zeros)
    return _epilogue(parts, h_bias.reshape(_NC, 1, _H))


# speedup vs baseline: 18.3190x; 1.5026x over previous
import jax
import jax.numpy as jnp
from jax import lax
from jax.experimental import pallas as pl
from jax.experimental.pallas import tpu as pltpu
from jax.experimental.pallas import tpu_sc as plsc

_N = 10000
_E = 320000
_F = 128
_H = _F // 2
_R = 8

_NC = 2
_NS = 16
_EPT = _E // _NS
_GCH = 200
_NG = _EPT // _GCH
_SCH = 100
_NSUB = _GCH // _SCH
_NP = 10240
_RPT = _NP // _NS


def _mm_body(x_ref, w_ref, o_ref):
    o_ref[...] = jnp.dot(x_ref[...], w_ref[0, 0],
                         preferred_element_type=jnp.float32)


def _rel_transform(x, w_halves):
    bn = 2000
    nb = _N // bn
    return pl.pallas_call(
        _mm_body,
        grid=(nb, _R, _NC),
        in_specs=[
            pl.BlockSpec((bn, _F), lambda i, r, h: (i, 0)),
            pl.BlockSpec((1, 1, _F, _H), lambda i, r, h: (h, r, 0, 0)),
        ],
        out_specs=pl.BlockSpec(
            (bn, _H), lambda i, r, h: (h * (_R * nb) + r * nb + i, 0)),
        out_shape=jax.ShapeDtypeStruct((_NC * _R * _N, _H), jnp.float32),
    )(x, w_halves)


def _gi_body(et_ref, src_ref, o_ref):
    g = et_ref[...] * _N + src_ref[...]
    o_ref[0] = g
    o_ref[1] = g + _R * _N


def _gather_indices(et2d, src2d):
    rows = _E // _F
    return pl.pallas_call(
        _gi_body,
        grid=(1,),
        in_specs=[
            pl.BlockSpec((rows, _F), lambda i: (0, 0)),
            pl.BlockSpec((rows, _F), lambda i: (0, 0)),
        ],
        out_specs=pl.BlockSpec((2, rows, _F), lambda i: (0, 0, 0)),
        out_shape=jax.ShapeDtypeStruct((2, rows, _F), jnp.int32),
    )(et2d, src2d)


def _sc_body(xr_hbm, gi_hbm, dst_hbm, zeros_hbm, out_hbm,
             gi_v, dst_v, rows0_v, rows1_v, agg_s, sem0, sem1):
    c = lax.axis_index("c")
    s = lax.axis_index("s")

    pltpu.sync_copy(gi_hbm.at[c, s], gi_v)
    pltpu.sync_copy(dst_hbm.at[s], dst_v)

    rows = pl.ds(s * _RPT, _RPT)
    pltpu.sync_copy(zeros_hbm.at[rows], agg_s.at[rows])

    plsc.subcore_barrier()

    bufs = (rows0_v, rows1_v)
    sems = (sem0, sem1)

    def _gather_start(m, b):
        pltpu.async_copy(xr_hbm.at[gi_v.at[pl.ds(m * _GCH, _GCH)]],
                         bufs[b], sems[b])

    def _gather_wait(b):
        pltpu.make_async_copy(xr_hbm.at[gi_v.at[pl.ds(0, _GCH)]],
                              bufs[b], sems[b]).wait()

    _gather_start(0, 0)

    def _pair(t, carry):
        for b in range(2):
            m = t * 2 + b
            _gather_wait(b)
            nxt = m + 1

            @pl.when(nxt < _NG)
            def _():
                _gather_start(nxt, (b + 1) % 2)

            for k in range(_NSUB):
                pltpu.sync_copy(bufs[b].at[pl.ds(k * _SCH, _SCH)],
                                agg_s.at[dst_v.at[m * _NSUB + k]], add=True)
        return carry
    lax.fori_loop(0, _NG // 2, _pair, 0)

    plsc.subcore_barrier()

    pltpu.sync_copy(agg_s.at[rows], out_hbm.at[c, rows])


_sc_scatter = pl.kernel(
    _sc_body,
    out_type=jax.ShapeDtypeStruct((_NC, _NP, _H), jnp.float32),
    mesh=plsc.VectorSubcoreMesh(core_axis_name="c", subcore_axis_name="s",
                                num_cores=_NC, num_subcores=_NS),
    scratch_types=[
        pltpu.VMEM((_EPT,), jnp.int32),
        pltpu.VMEM((_EPT // _SCH, _SCH), jnp.int32),
        pltpu.VMEM((_GCH, _H), jnp.float32),
        pltpu.VMEM((_GCH, _H), jnp.float32),
        pltpu.VMEM_SHARED((_NP, _H), jnp.float32),
        pltpu.SemaphoreType.DMA,
        pltpu.SemaphoreType.DMA,
    ],
    compiler_params=pltpu.CompilerParams(use_tc_tiling_on_sc=False),
)


def _ep_body(p_ref, b_ref, o_ref):
    full = jnp.concatenate([p_ref[0], p_ref[1]], axis=1)
    bias = jnp.concatenate([b_ref[0], b_ref[1]], axis=1)
    o_ref[...] = full + bias


def _epilogue(parts, bias2d):
    bn = 2000
    return pl.pallas_call(
        _ep_body,
        grid=(_N // bn,),
        in_specs=[
            pl.BlockSpec((_NC, bn, _H), lambda i: (0, i, 0)),
            pl.BlockSpec((_NC, 1, _H), lambda i: (0, 0, 0)),
        ],
        out_specs=pl.BlockSpec((bn, _F), lambda i: (i, 0)),
        out_shape=jax.ShapeDtypeStruct((_N, _F), jnp.float32),
    )(parts, bias2d)


def kernel(x, edge_index, edge_type, weight, h_bias):
    w_halves = weight.reshape(_R, _F, _NC, _H).transpose(2, 0, 1, 3)
    xr = _rel_transform(x, w_halves)
    et2d = edge_type.reshape(_E // _F, _F)
    src2d = edge_index[0].reshape(_E // _F, _F)
    gi = _gather_indices(et2d, src2d).reshape(_NC, _NS, _EPT)
    dst = edge_index[1].reshape(_NS, _EPT // _SCH, _SCH)
    zeros = jnp.zeros((_NP, _H), jnp.float32)
    parts = _sc_scatter(xr, gi, dst, zeros)
    return _epilogue(parts, h_bias.reshape(_NC, 1, _H))


# speedup vs baseline: 25.8054x; 1.4087x over previous
import jax
import jax.numpy as jnp
from jax import lax
from jax.experimental import pallas as pl
from jax.experimental.pallas import tpu as pltpu
from jax.experimental.pallas import tpu_sc as plsc

_N = 10000
_E = 320000
_F = 128
_H = _F // 2
_R = 8

_NC = 2
_NS = 16
_EPT = _E // _NS
_GCH = 200
_NG = _EPT // _GCH
_SCH = 100
_NSUB = _GCH // _SCH
_NP = 10240
_RPT = _NP // _NS


def _mm_body(x_ref, w_ref, o_ref):
    o_ref[...] = jnp.dot(x_ref[...], w_ref[0],
                         preferred_element_type=jnp.float32)


def _rel_transform(x, weight):
    bn = 2000
    nb = _N // bn
    return pl.pallas_call(
        _mm_body,
        grid=(nb, _R),
        in_specs=[
            pl.BlockSpec((bn, _F), lambda i, r: (i, 0)),
            pl.BlockSpec((1, _F, _F), lambda i, r: (r, 0, 0)),
        ],
        out_specs=pl.BlockSpec((bn, _F), lambda i, r: (r * nb + i, 0)),
        out_shape=jax.ShapeDtypeStruct((_R * _N, _F), jnp.float32),
    )(x, weight)


def _gi_body(et_ref, src_ref, o_ref):
    g = (et_ref[...] * _N + src_ref[...]) * 2
    o_ref[0] = g
    o_ref[1] = g + 1


def _gather_indices(et2d, src2d):
    rows = _E // _F
    return pl.pallas_call(
        _gi_body,
        grid=(1,),
        in_specs=[
            pl.BlockSpec((rows, _F), lambda i: (0, 0)),
            pl.BlockSpec((rows, _F), lambda i: (0, 0)),
        ],
        out_specs=pl.BlockSpec((2, rows, _F), lambda i: (0, 0, 0)),
        out_shape=jax.ShapeDtypeStruct((2, rows, _F), jnp.int32),
    )(et2d, src2d)


def _sc_body(xr_hbm, gi_hbm, dst_hbm, zeros_hbm, out_hbm,
             gi_v, dst_v, rows0_v, rows1_v, agg_s, sem0, sem1):
    c = lax.axis_index("c")
    s = lax.axis_index("s")

    pltpu.sync_copy(gi_hbm.at[c, s], gi_v)
    pltpu.sync_copy(dst_hbm.at[s], dst_v)

    rows = pl.ds(s * _RPT, _RPT)
    pltpu.sync_copy(zeros_hbm.at[rows], agg_s.at[rows])

    plsc.subcore_barrier()

    bufs = (rows0_v, rows1_v)
    sems = (sem0, sem1)

    def _gather_start(m, b):
        pltpu.async_copy(xr_hbm.at[gi_v.at[pl.ds(m * _GCH, _GCH)]],
                         bufs[b], sems[b])

    def _gather_wait(b):
        pltpu.make_async_copy(xr_hbm.at[gi_v.at[pl.ds(0, _GCH)]],
                              bufs[b], sems[b]).wait()

    _gather_start(0, 0)

    def _pair(t, carry):
        for b in range(2):
            m = t * 2 + b
            _gather_wait(b)
            nxt = m + 1

            @pl.when(nxt < _NG)
            def _():
                _gather_start(nxt, (b + 1) % 2)

            for k in range(_NSUB):
                pltpu.sync_copy(bufs[b].at[pl.ds(k * _SCH, _SCH)],
                                agg_s.at[dst_v.at[m * _NSUB + k]], add=True)
        return carry
    lax.fori_loop(0, _NG // 2, _pair, 0)

    plsc.subcore_barrier()

    pltpu.sync_copy(agg_s.at[rows], out_hbm.at[c, rows])


_sc_scatter = pl.kernel(
    _sc_body,
    out_type=jax.ShapeDtypeStruct((_NC, _NP, _H), jnp.float32),
    mesh=plsc.VectorSubcoreMesh(core_axis_name="c", subcore_axis_name="s",
                                num_cores=_NC, num_subcores=_NS),
    scratch_types=[
        pltpu.VMEM((_EPT,), jnp.int32),
        pltpu.VMEM((_EPT // _SCH, _SCH), jnp.int32),
        pltpu.VMEM((_GCH, _H), jnp.float32),
        pltpu.VMEM((_GCH, _H), jnp.float32),
        pltpu.VMEM_SHARED((_NP, _H), jnp.float32),
        pltpu.SemaphoreType.DMA,
        pltpu.SemaphoreType.DMA,
    ],
    compiler_params=pltpu.CompilerParams(use_tc_tiling_on_sc=False),
)


def _ep_body(p_ref, b_ref, o_ref):
    full = jnp.concatenate([p_ref[0], p_ref[1]], axis=1)
    bias = jnp.concatenate([b_ref[0], b_ref[1]], axis=1)
    o_ref[...] = full + bias


def _epilogue(parts, bias2d):
    bn = 2000
    return pl.pallas_call(
        _ep_body,
        grid=(_N // bn,),
        in_specs=[
            pl.BlockSpec((_NC, bn, _H), lambda i: (0, i, 0)),
            pl.BlockSpec((_NC, 1, _H), lambda i: (0, 0, 0)),
        ],
        out_specs=pl.BlockSpec((bn, _F), lambda i: (i, 0)),
        out_shape=jax.ShapeDtypeStruct((_N, _F), jnp.float32),
    )(parts, bias2d)


def kernel(x, edge_index, edge_type, weight, h_bias):
    xr = _rel_transform(x, weight).reshape(_NC * _R * _N, _H)
    et2d = edge_type.reshape(_E // _F, _F)
    src2d = edge_index[0].reshape(_E // _F, _F)
    gi = _gather_indices(et2d, src2d).reshape(_NC, _NS, _EPT)
    dst = edge_index[1].reshape(_NS, _EPT // _SCH, _SCH)
    zeros = jnp.zeros((_NP, _H), jnp.float32)
    parts = _sc_scatter(xr, gi, dst, zeros)
    return _epilogue(parts, h_bias.reshape(_NC, 1, _H))
